# Initial kernel scaffold; baseline (speedup 1.0000x reference)
import jax
import jax.numpy as jnp
from jax.experimental import pallas as pl


def kernel(ui_adj, social_adj, user_emb, item_emb, W_ui0, W_ui1, W_s0, W_s1):
    raise NotImplementedError("write your pallas kernel here")


# speedup vs baseline: 5.1748x; 5.1748x over previous
import functools

import jax
import jax.numpy as jnp
from jax import lax
from jax.experimental import pallas as pl
from jax.experimental.pallas import tpu as pltpu
from jax.experimental.pallas import tpu_sc as plsc

_F32 = jnp.float32
_LANES = 128
_CH = 4
_NTILES = 16
_ZROWS = 1000


def _leaky(x):
    return jnp.where(x >= 0, x, 0.5 * x)


@functools.lru_cache(maxsize=None)
def _make_sc_agg(n, nz, g):
    gpt = g // _NTILES
    nchunk = gpt // _CH
    mesh = plsc.VectorSubcoreMesh(core_axis_name="c", subcore_axis_name="s")

    @functools.partial(
        pl.kernel,
        mesh=mesh,
        compiler_params=pltpu.CompilerParams(use_tc_tiling_on_sc=False),
        out_type=[jax.ShapeDtypeStruct((n, 32), _F32),
                  jax.ShapeDtypeStruct((n, 32), _F32)],
        scratch_types=[
            pltpu.VMEM((_CH, _LANES), jnp.int32),
            pltpu.VMEM((_CH, _LANES), jnp.int32),
            pltpu.VMEM((_CH * _LANES, 32), _F32),
            pltpu.VMEM_SHARED((nz, 32), _F32),
            pltpu.SemaphoreType.DMA,
        ],
    )
    def agg(nf0, nf1, col2, row2, zrows, out0, out1,
            colv, rowv, rowsv, acc, sem):
        c = lax.axis_index("c")
        s = lax.axis_index("s")

        def zb(z, carry):
            @pl.when(lax.rem(z, _NTILES) == s)
            def _():
                pltpu.sync_copy(zrows, acc.at[pl.ds(z * _ZROWS, _ZROWS)])
            return carry
        lax.fori_loop(0, nz // _ZROWS, zb, 0)
        plsc.subcore_barrier()

        def edges(nf):
            def cb(i, carry):
                g0 = s * gpt + i * _CH
                pltpu.sync_copy(col2.at[pl.ds(g0, _CH)], colv)
                pltpu.sync_copy(row2.at[pl.ds(g0, _CH)], rowv)
                hs = [pltpu.async_copy(nf.at[colv.at[j]],
                                       rowsv.at[pl.ds(j * _LANES, _LANES)],
                                       sem)
                      for j in range(_CH)]
                for h in hs:
                    h.wait()
                for j in range(_CH):
                    pltpu.sync_copy(rowsv.at[pl.ds(j * _LANES, _LANES)],
                                    acc.at[rowv.at[j]], add=True)
                return carry
            lax.fori_loop(0, nchunk, cb, 0)

        @pl.when(c == 0)
        def _():
            edges(nf0)

        @pl.when(c == 1)
        def _():
            edges(nf1)

        plsc.subcore_barrier()

        def drain(out):
            def db(d, carry):
                @pl.when(lax.rem(d, _NTILES) == s)
                def _():
                    pltpu.sync_copy(acc.at[pl.ds(d * _ZROWS, _ZROWS)],
                                    out.at[pl.ds(d * _ZROWS, _ZROWS)])
                return carry
            lax.fori_loop(0, n // _ZROWS, db, 0)

        @pl.when(c == 0)
        def _():
            drain(out0)

        @pl.when(c == 1)
        def _():
            drain(out1)

    return agg


_BN = 1000


def _mm_split(emb, w):
    n = emb.shape[0]

    def body(e_ref, w_ref, o0_ref, o1_ref):
        p = jnp.dot(e_ref[...], w_ref[...], preferred_element_type=_F32)
        o0_ref[...] = p[:, :32]
        o1_ref[...] = p[:, 32:]

    return pl.pallas_call(
        body,
        grid=(n // _BN,),
        in_specs=[pl.BlockSpec((_BN, 64), lambda i: (i, 0)),
                  pl.BlockSpec((64, 64), lambda i: (0, 0))],
        out_specs=[pl.BlockSpec((_BN, 32), lambda i: (i, 0)),
                   pl.BlockSpec((_BN, 32), lambda i: (i, 0))],
        out_shape=[jax.ShapeDtypeStruct((n, 32), _F32),
                   jax.ShapeDtypeStruct((n, 32), _F32)],
    )(emb, w)


def _mid_layer(a0, a1, w):
    n = a0.shape[0]

    def body(a0_ref, a1_ref, w_ref, on_ref, o0_ref, o1_ref):
        e = jnp.concatenate([_leaky(a0_ref[...]), _leaky(a1_ref[...])], axis=1)
        nrm = jnp.sqrt(jnp.sum(e * e, axis=1, keepdims=True))
        on_ref[...] = e / jnp.maximum(nrm, 1e-12)
        p = jnp.dot(e, w_ref[...], preferred_element_type=_F32)
        o0_ref[...] = p[:, :32]
        o1_ref[...] = p[:, 32:]

    return pl.pallas_call(
        body,
        grid=(n // _BN,),
        in_specs=[pl.BlockSpec((_BN, 32), lambda i: (i, 0)),
                  pl.BlockSpec((_BN, 32), lambda i: (i, 0)),
                  pl.BlockSpec((64, 64), lambda i: (0, 0))],
        out_specs=[pl.BlockSpec((_BN, 64), lambda i: (i, 0)),
                   pl.BlockSpec((_BN, 32), lambda i: (i, 0)),
                   pl.BlockSpec((_BN, 32), lambda i: (i, 0))],
        out_shape=[jax.ShapeDtypeStruct((n, 64), _F32),
                   jax.ShapeDtypeStruct((n, 32), _F32),
                   jax.ShapeDtypeStruct((n, 32), _F32)],
    )(a0, a1, w)


def _final_layer(a0, a1, base, n1):
    n = a0.shape[0]

    def body(a0_ref, a1_ref, b_ref, n1_ref, o_ref):
        e = jnp.concatenate([_leaky(a0_ref[...]), _leaky(a1_ref[...])], axis=1)
        nrm = jnp.sqrt(jnp.sum(e * e, axis=1, keepdims=True))
        o_ref[...] = b_ref[...] + n1_ref[...] + e / jnp.maximum(nrm, 1e-12)

    return pl.pallas_call(
        body,
        grid=(n // _BN,),
        in_specs=[pl.BlockSpec((_BN, 32), lambda i: (i, 0)),
                  pl.BlockSpec((_BN, 32), lambda i: (i, 0)),
                  pl.BlockSpec((_BN, 64), lambda i: (i, 0)),
                  pl.BlockSpec((_BN, 64), lambda i: (i, 0))],
        out_specs=pl.BlockSpec((_BN, 64), lambda i: (i, 0)),
        out_shape=jax.ShapeDtypeStruct((n, 64), _F32),
    )(a0, a1, base, n1)


def _prep_edges(adj, e_pad, trash):
    row = adj[0].astype(jnp.int32)
    col = adj[1].astype(jnp.int32)
    pad = e_pad - row.shape[0]
    col_p = jnp.concatenate([col, jnp.zeros((pad,), jnp.int32)])
    row_p = jnp.concatenate([row, jnp.full((pad,), trash, jnp.int32)])
    return col_p.reshape(-1, _LANES), row_p.reshape(-1, _LANES)


def _gcn_stack(emb0, w_a, w_b, col2, row2, zrows, nz):
    n = emb0.shape[0]
    g = col2.shape[0]
    agg = _make_sc_agg(n, nz, g)
    nf0, nf1 = _mm_split(emb0, w_a)
    a0, a1 = agg(nf0, nf1, col2, row2, zrows)
    n1, nf0b, nf1b = _mid_layer(a0, a1, w_b)
    b0, b1 = agg(nf0b, nf1b, col2, row2, zrows)
    return _final_layer(b0, b1, emb0, n1)


def kernel(ui_adj, social_adj, user_emb, item_emb, W_ui0, W_ui1, W_s0, W_s1):
    n_user = user_emb.shape[0]
    n_ui = n_user + item_emb.shape[0]

    e0 = jnp.concatenate([user_emb, item_emb], axis=0)
    zrows = jnp.zeros((_ZROWS, 32), _F32)

    ui_col, ui_row = _prep_edges(ui_adj, 802816, n_ui)
    ui_emb = _gcn_stack(e0, W_ui0, W_ui1, ui_col, ui_row, zrows, 51000)

    s_col, s_row = _prep_edges(social_adj, 409600, n_user)
    social_emb = _gcn_stack(user_emb, W_s0, W_s1, s_col, s_row, zrows, 26000)

    return (ui_emb, social_emb)
